# baseline (speedup 1.0000x reference)
import jax
import jax.numpy as jnp
from jax import lax
from jax.experimental import pallas as pl
from jax.experimental.pallas import tpu as pltpu
from jax.experimental.pallas import tpu_sc as plsc

_M = 1024
_D = 16
_N = 100000
_NPAD = 100352
_BN = 2048
_K = 16
_NW = 32
_QPW = _M // _NW
_C = 3584
_NCH = _NPAD // _C
_GPC = _C // 16
_VPC = _GPC // 16
_GW = _NPAD // 16
_PADVAL = 1e18


_BM = 256


def _proxy_body(qm2_ref, s_ref, out_ref, min_ref):
  s = s_ref[...]
  sn = jnp.sum(s * s, axis=1)
  acc = lax.dot_general(qm2_ref[...], s, (((1,), (1,)), ((), ())),
                        preferred_element_type=jnp.float32)
  prox = acc + sn[None, :]
  out_ref[...] = prox
  min_ref[...] = jnp.min(prox.reshape(_BM, _BN // 16, 16), axis=-1)


def _compute_proxy(qm2, spad):
  return pl.pallas_call(
      _proxy_body,
      grid=(_M // _BM, _NPAD // _BN),
      in_specs=[
          pl.BlockSpec((_BM, _D), lambda m, n: (m, 0)),
          pl.BlockSpec((_BN, _D), lambda m, n: (n, 0)),
      ],
      out_specs=[
          pl.BlockSpec((_BM, _BN), lambda m, n: (m, n)),
          pl.BlockSpec((_BM, _BN // 16), lambda m, n: (m, n)),
      ],
      out_shape=[
          jax.ShapeDtypeStruct((_M, _NPAD), jnp.float32),
          jax.ShapeDtypeStruct((_M, _NPAD // 16), jnp.float32),
      ],
  )(qm2, spad)


def _merge16(bv, bi, cv, ci):
  cs, cis = plsc.sort_key_val(cv, ci)
  cr = lax.rev(cs, (0,))
  cir = lax.rev(cis, (0,))
  take = cr < bv
  nv = jnp.where(take, cr, bv)
  ni = jnp.where(take, cir, bi)
  return plsc.sort_key_val(nv, ni)


def _scan_chunk(pbuf, gbuf, chunk_i, carry):
  iota = lax.iota(jnp.int32, 16)
  cbase = chunk_i * _C
  gbase = chunk_i * _GPC
  out = list(carry)
  for r in range(8):
    bv, bi, thr = carry[3 * r], carry[3 * r + 1], carry[3 * r + 2]
    gs = [gbuf[r, pl.ds(gbase + v * 16, 16)] for v in range(_VPC)]
    while len(gs) > 1:
      gs = [jnp.minimum(gs[k], gs[k + 1]) if k + 1 < len(gs) else gs[k]
            for k in range(0, len(gs), 2)]
    hit = jnp.min(gs[0]) < thr

    def drill(bv, bi, thr, r=r):
      def vec(v, st):
        bv, bi, thr = st
        g = gbuf[r, pl.ds(gbase + v * 16, 16)]
        vhit = jnp.min(g) < thr

        def do_vec(bv, bi, thr):
          def group(j, st2):
            bv, bi, thr = st2
            vals = pbuf[r, pl.ds(v * 256 + j * 16, 16)]
            ghit = jnp.min(vals) < thr

            def do_merge(bv, bi, thr):
              ci = cbase + v * 256 + j * 16 + iota
              bv, bi = _merge16(bv, bi, vals, ci)
              return bv, bi, bv[15]

            return lax.cond(ghit, do_merge,
                            lambda bv, bi, thr: (bv, bi, thr), bv, bi, thr)

          return lax.fori_loop(0, 16, group, (bv, bi, thr))

        return lax.cond(vhit, do_vec,
                        lambda bv, bi, thr: (bv, bi, thr), bv, bi, thr)

      return lax.fori_loop(0, _VPC, vec, (bv, bi, thr))

    nb = lax.cond(hit, drill, lambda bv, bi, thr: (bv, bi, thr),
                  bv, bi, thr)
    out[3 * r], out[3 * r + 1], out[3 * r + 2] = nb
  return tuple(out)


def _topk_body(proxy, gmin, d2_out, idx_out,
               pbuf0, pbuf1, gbuf, res_v, resi_v,
               sem_a, sem_b, sem_g):
  c = lax.axis_index("c")
  s = lax.axis_index("s")
  wid = s * 2 + c
  qbase = wid * _QPW

  def octet(o, _):
    rbase = qbase + o * 8

    def pslab(cb):
      return proxy.at[pl.ds(rbase, 8), pl.ds(cb, _C)]

    gslab = gmin.at[pl.ds(rbase, 8), :]
    pltpu.async_copy(pslab(0), pbuf0, sem_a)
    pltpu.async_copy(gslab, gbuf, sem_g)

    init = []
    for _r in range(8):
      init += [jnp.full((16,), jnp.inf, jnp.float32),
               jnp.zeros((16,), jnp.int32), jnp.float32(jnp.inf)]

    pltpu.make_async_copy(gslab, gbuf, sem_g).wait()

    def pair(i, carry):
      c0 = 2 * i
      pltpu.async_copy(pslab((c0 + 1) * _C), pbuf1, sem_b)
      pltpu.make_async_copy(pslab(c0 * _C), pbuf0, sem_a).wait()
      carry = _scan_chunk(pbuf0, gbuf, c0, carry)

      @pl.when(i < _NCH // 2 - 1)
      def _():
        pltpu.async_copy(pslab((c0 + 2) * _C), pbuf0, sem_a)

      pltpu.make_async_copy(pslab((c0 + 1) * _C), pbuf1, sem_b).wait()
      carry = _scan_chunk(pbuf1, gbuf, c0 + 1, carry)
      return carry

    carry = lax.fori_loop(0, _NCH // 2, pair, tuple(init))

    for r in range(8):
      res_v[r] = carry[3 * r]
      resi_v[r] = carry[3 * r + 1]
    pltpu.sync_copy(res_v, d2_out.at[pl.ds(rbase, 8)])
    pltpu.sync_copy(resi_v, idx_out.at[pl.ds(rbase, 8)])
    return 0

  lax.fori_loop(0, _QPW // 8, octet, 0)


def _topk(proxy, gmin):
  mesh = plsc.VectorSubcoreMesh(core_axis_name="c", subcore_axis_name="s")
  f = pl.kernel(
      _topk_body,
      out_type=(
          jax.ShapeDtypeStruct((_M, _K), jnp.float32),
          jax.ShapeDtypeStruct((_M, _K), jnp.int32),
      ),
      mesh=mesh,
      scratch_types=[
          pltpu.VMEM((8, _C), jnp.float32),
          pltpu.VMEM((8, _C), jnp.float32),
          pltpu.VMEM((8, _GW), jnp.float32),
          pltpu.VMEM((8, _K), jnp.float32),
          pltpu.VMEM((8, _K), jnp.int32),
          pltpu.SemaphoreType.DMA,
          pltpu.SemaphoreType.DMA,
          pltpu.SemaphoreType.DMA,
      ],
      compiler_params=pltpu.CompilerParams(needs_layout_passes=False),
  )
  return f(proxy, gmin)


def _sqrt_body(bv_ref, q_ref, out_ref):
  q = q_ref[...]
  qn = jnp.sum(q * q, axis=1, keepdims=True)
  out_ref[...] = jnp.sqrt(jnp.maximum(bv_ref[...] + qn, 0.0))


def _sqrt(bv, q):
  return pl.pallas_call(
      _sqrt_body,
      out_shape=jax.ShapeDtypeStruct((_M, _K), jnp.float32),
  )(bv, q)


def kernel(query, support):
  q = query[0]
  s = support[0]
  qm2 = -2.0 * q
  spad = jnp.pad(s, ((0, _NPAD - _N), (0, 0)), constant_values=_PADVAL)
  proxy, gmin = _compute_proxy(qm2, spad)
  bv, idx = _topk(proxy, gmin)
  values = _sqrt(bv, q)
  return (values.reshape(1, _M, _K), idx.reshape(1, _M, _K))


# speedup vs baseline: 1.0077x; 1.0077x over previous
import jax
import jax.numpy as jnp
from jax import lax
from jax.experimental import pallas as pl
from jax.experimental.pallas import tpu as pltpu
from jax.experimental.pallas import tpu_sc as plsc

_M = 1024
_D = 16
_N = 100000
_NPAD = 100352
_BN = 2048
_K = 16
_NW = 32
_QPW = _M // _NW
_C = 3584
_NCH = _NPAD // _C
_GPC = _C // 16
_VPC = _GPC // 16
_GW = _NPAD // 16
_PADVAL = 1e18


_BM = 256


def _proxy_body(qm2_ref, s_ref, out_ref, min_ref):
  s = s_ref[...]
  sn = jnp.sum(s * s, axis=1)
  acc = lax.dot_general(qm2_ref[...], s, (((1,), (1,)), ((), ())),
                        preferred_element_type=jnp.float32)
  prox = acc + sn[None, :]
  out_ref[...] = prox
  min_ref[...] = jnp.min(prox.reshape(_BM, _BN // 16, 16), axis=-1)


def _compute_proxy(qm2, spad):
  return pl.pallas_call(
      _proxy_body,
      grid=(_M // _BM, _NPAD // _BN),
      in_specs=[
          pl.BlockSpec((_BM, _D), lambda m, n: (m, 0)),
          pl.BlockSpec((_BN, _D), lambda m, n: (n, 0)),
      ],
      out_specs=[
          pl.BlockSpec((_BM, _BN), lambda m, n: (m, n)),
          pl.BlockSpec((_BM, _BN // 16), lambda m, n: (m, n)),
      ],
      out_shape=[
          jax.ShapeDtypeStruct((_M, _NPAD), jnp.float32),
          jax.ShapeDtypeStruct((_M, _NPAD // 16), jnp.float32),
      ],
  )(qm2, spad)


def _merge16(bv, bi, cv, ci):
  cs, cis = plsc.sort_key_val(cv, ci)
  cr = lax.rev(cs, (0,))
  cir = lax.rev(cis, (0,))
  take = cr < bv
  nv = jnp.where(take, cr, bv)
  ni = jnp.where(take, cir, bi)
  return plsc.sort_key_val(nv, ni)


def _scan_chunk(pbuf, gbuf, chunk_i, carry):
  iota = lax.iota(jnp.int32, 16)
  cbase = chunk_i * _C
  gbase = chunk_i * _GPC

  def vecblock(v, carry):
    out = list(carry)
    for r in range(8):
      bv, bi, thr = carry[3 * r], carry[3 * r + 1], carry[3 * r + 2]
      g = gbuf[r, pl.ds(gbase + v * 16, 16)]
      hit = jnp.min(g) < thr

      def drill(bv, bi, thr, r=r):
        def group(j, st):
          bv, bi, thr = st
          vals = pbuf[r, pl.ds(v * 256 + j * 16, 16)]
          ghit = jnp.min(vals) < thr

          def do_merge(bv, bi, thr):
            ci = cbase + v * 256 + j * 16 + iota
            bv, bi = _merge16(bv, bi, vals, ci)
            return bv, bi, bv[15]

          return lax.cond(ghit, do_merge,
                          lambda bv, bi, thr: (bv, bi, thr), bv, bi, thr)

        return lax.fori_loop(0, 16, group, (bv, bi, thr))

      nb = lax.cond(hit, drill, lambda bv, bi, thr: (bv, bi, thr),
                    bv, bi, thr)
      out[3 * r], out[3 * r + 1], out[3 * r + 2] = nb
    return tuple(out)

  return lax.fori_loop(0, _VPC, vecblock, carry)


def _topk_body(proxy, gmin, d2_out, idx_out,
               pbuf0, pbuf1, gbuf, res_v, resi_v,
               sem_a, sem_b, sem_g):
  c = lax.axis_index("c")
  s = lax.axis_index("s")
  wid = s * 2 + c
  qbase = wid * _QPW

  def octet(o, _):
    rbase = qbase + o * 8

    def pslab(cb):
      return proxy.at[pl.ds(rbase, 8), pl.ds(cb, _C)]

    gslab = gmin.at[pl.ds(rbase, 8), :]
    pltpu.async_copy(pslab(0), pbuf0, sem_a)
    pltpu.async_copy(gslab, gbuf, sem_g)

    init = []
    for _r in range(8):
      init += [jnp.full((16,), jnp.inf, jnp.float32),
               jnp.zeros((16,), jnp.int32), jnp.float32(jnp.inf)]

    pltpu.make_async_copy(gslab, gbuf, sem_g).wait()

    def pair(i, carry):
      c0 = 2 * i
      pltpu.async_copy(pslab((c0 + 1) * _C), pbuf1, sem_b)
      pltpu.make_async_copy(pslab(c0 * _C), pbuf0, sem_a).wait()
      carry = _scan_chunk(pbuf0, gbuf, c0, carry)

      @pl.when(i < _NCH // 2 - 1)
      def _():
        pltpu.async_copy(pslab((c0 + 2) * _C), pbuf0, sem_a)

      pltpu.make_async_copy(pslab((c0 + 1) * _C), pbuf1, sem_b).wait()
      carry = _scan_chunk(pbuf1, gbuf, c0 + 1, carry)
      return carry

    carry = lax.fori_loop(0, _NCH // 2, pair, tuple(init))

    for r in range(8):
      res_v[r] = carry[3 * r]
      resi_v[r] = carry[3 * r + 1]
    pltpu.sync_copy(res_v, d2_out.at[pl.ds(rbase, 8)])
    pltpu.sync_copy(resi_v, idx_out.at[pl.ds(rbase, 8)])
    return 0

  lax.fori_loop(0, _QPW // 8, octet, 0)


def _topk(proxy, gmin):
  mesh = plsc.VectorSubcoreMesh(core_axis_name="c", subcore_axis_name="s")
  f = pl.kernel(
      _topk_body,
      out_type=(
          jax.ShapeDtypeStruct((_M, _K), jnp.float32),
          jax.ShapeDtypeStruct((_M, _K), jnp.int32),
      ),
      mesh=mesh,
      scratch_types=[
          pltpu.VMEM((8, _C), jnp.float32),
          pltpu.VMEM((8, _C), jnp.float32),
          pltpu.VMEM((8, _GW), jnp.float32),
          pltpu.VMEM((8, _K), jnp.float32),
          pltpu.VMEM((8, _K), jnp.int32),
          pltpu.SemaphoreType.DMA,
          pltpu.SemaphoreType.DMA,
          pltpu.SemaphoreType.DMA,
      ],
      compiler_params=pltpu.CompilerParams(needs_layout_passes=False),
  )
  return f(proxy, gmin)


def _sqrt_body(bv_ref, q_ref, out_ref):
  q = q_ref[...]
  qn = jnp.sum(q * q, axis=1, keepdims=True)
  out_ref[...] = jnp.sqrt(jnp.maximum(bv_ref[...] + qn, 0.0))


def _sqrt(bv, q):
  return pl.pallas_call(
      _sqrt_body,
      out_shape=jax.ShapeDtypeStruct((_M, _K), jnp.float32),
  )(bv, q)


def kernel(query, support):
  q = query[0]
  s = support[0]
  qm2 = -2.0 * q
  spad = jnp.pad(s, ((0, _NPAD - _N), (0, 0)), constant_values=_PADVAL)
  proxy, gmin = _compute_proxy(qm2, spad)
  bv, idx = _topk(proxy, gmin)
  values = _sqrt(bv, q)
  return (values.reshape(1, _M, _K), idx.reshape(1, _M, _K))


# speedup vs baseline: 2.9263x; 2.9040x over previous
import jax
import jax.numpy as jnp
from jax import lax
from jax.experimental import pallas as pl
from jax.experimental.pallas import tpu as pltpu
from jax.experimental.pallas import tpu_sc as plsc

_M = 1024
_D = 16
_N = 100000
_NPAD = 100352
_BN = 2048
_BM = 256
_K = 16
_NW = 32
_QPW = _M // _NW
_NG = _NPAD // 16
_NBLK = _NG // 128
_NS = _NPAD // 128
_PADVAL = 1e18
_QCAP = 128


def _proxy_body(qm2_ref, s_ref, sp_ref, out_ref, min_ref):
  s = s_ref[...]
  sn = jnp.sum(s * s, axis=1)
  acc = lax.dot_general(qm2_ref[...], s, (((1,), (1,)), ((), ())),
                        preferred_element_type=jnp.float32)
  out_ref[...] = acc + sn[None, :]
  sp = sp_ref[...]
  snp = jnp.sum(sp * sp, axis=1)
  accp = lax.dot_general(qm2_ref[...], sp, (((1,), (1,)), ((), ())),
                         preferred_element_type=jnp.float32)
  proxp = accp + snp[None, :]
  min_ref[...] = jnp.min(proxp.reshape(_BM, 16, _BN // 16), axis=1)


def _compute_proxy(qm2, spad, sperm):
  return pl.pallas_call(
      _proxy_body,
      grid=(_M // _BM, _NPAD // _BN),
      in_specs=[
          pl.BlockSpec((_BM, _D), lambda m, n: (m, 0)),
          pl.BlockSpec((_BN, _D), lambda m, n: (n, 0)),
          pl.BlockSpec((_BN, _D), lambda m, n: (n, 0)),
      ],
      out_specs=[
          pl.BlockSpec((_BM, _BN), lambda m, n: (m, n)),
          pl.BlockSpec((_BM, _BN // 16), lambda m, n: (m, n)),
      ],
      out_shape=[
          jax.ShapeDtypeStruct((_M, _NPAD), jnp.float32),
          jax.ShapeDtypeStruct((_M, _NG), jnp.float32),
      ],
  )(qm2, spad, sperm)


def _merge16(bv, bi, cv, ci):
  cs, cis = plsc.sort_key_val(cv, ci)
  cr = lax.rev(cs, (0,))
  cir = lax.rev(cis, (0,))
  take = cr < bv
  nv = jnp.where(take, cr, bv)
  ni = jnp.where(take, cir, bi)
  return plsc.sort_key_val(nv, ni)


def _topk_body(proxy, gmin, slack2d, d2_out, idx_out,
               gbuf, qlist, sbuf, fbuf0, fbuf1, res_v, resi_v,
               sem_g, sem_s, sem_f0, sem_f1):
  c = lax.axis_index("c")
  s = lax.axis_index("s")
  wid = s * 2 + c
  qbase = wid * _QPW
  iota = lax.iota(jnp.int32, 16)
  inf = jnp.float32(jnp.inf)

  def octet(o, _):
    rbase = qbase + o * 8
    gslab = gmin.at[pl.ds(rbase, 8), :]
    sslab = slack2d.at[pl.ds(rbase, 8), :]
    pltpu.async_copy(gslab, gbuf, sem_g)
    pltpu.async_copy(sslab, sbuf, sem_s)
    pltpu.make_async_copy(gslab, gbuf, sem_g).wait()
    pltpu.make_async_copy(sslab, sbuf, sem_s).wait()

    init1 = []
    for _r in range(8):
      init1 += [jnp.full((16,), jnp.inf, jnp.float32),
                jnp.zeros((16,), jnp.int32), inf]

    def block1(b, carry):
      base = b * 128
      out = list(carry)
      for r in range(8):
        gv, gi, thr = carry[3 * r], carry[3 * r + 1], carry[3 * r + 2]
        vs = [gbuf[r, pl.ds(base + j * 16, 16)] for j in range(8)]
        m01 = jnp.minimum(vs[0], vs[1])
        m23 = jnp.minimum(vs[2], vs[3])
        m45 = jnp.minimum(vs[4], vs[5])
        m67 = jnp.minimum(vs[6], vs[7])
        m = jnp.minimum(jnp.minimum(m01, m23), jnp.minimum(m45, m67))
        hit = jnp.min(m) < thr

        def do_block(gv, gi, thr, r=r, base=base):
          def group(j, st):
            gv, gi, thr = st
            vals = gbuf[r, pl.ds(base + j * 16, 16)]
            ghit = jnp.min(vals) < thr

            def do_merge(gv, gi, thr):
              ci = base + j * 16 + iota
              gv, gi = _merge16(gv, gi, vals, ci)
              return gv, gi, gv[15]

            return lax.cond(ghit, do_merge,
                            lambda gv, gi, thr: (gv, gi, thr), gv, gi, thr)

          return lax.fori_loop(0, 8, group, (gv, gi, thr))

        nb = lax.cond(hit, do_block, lambda gv, gi, thr: (gv, gi, thr),
                      gv, gi, thr)
        out[3 * r], out[3 * r + 1], out[3 * r + 2] = nb
      return tuple(out)

    carry1 = lax.fori_loop(0, _NBLK, block1, tuple(init1))
    thr0s = [carry1[3 * r + 2] + 2.0 * sbuf[r, pl.ds(0, 16)][0]
             for r in range(8)]

    def block2(b, carry):
      base = b * 128
      out = list(carry)
      for r in range(8):
        cnt = carry[r]
        thr0 = thr0s[r]
        vs = [gbuf[r, pl.ds(base + j * 16, 16)] for j in range(8)]
        m01 = jnp.minimum(vs[0], vs[1])
        m23 = jnp.minimum(vs[2], vs[3])
        m45 = jnp.minimum(vs[4], vs[5])
        m67 = jnp.minimum(vs[6], vs[7])
        m = jnp.minimum(jnp.minimum(m01, m23), jnp.minimum(m45, m67))
        hit = jnp.min(m) <= thr0

        def do_block(cnt, r=r, base=base, thr0=thr0):
          def gv_loop(j, cnt):
            g = gbuf[r, pl.ds(base + j * 16, 16)]
            h0 = jnp.min(jnp.where(iota < 8, g, inf)) <= thr0
            h1 = jnp.min(jnp.where(iota >= 8, g, inf)) <= thr0
            sid = b * 16 + j * 2

            def app0(cnt):
              qlist[r, jnp.minimum(cnt, _QCAP - 1)] = sid
              return cnt + 1

            cnt = lax.cond(h0, app0, lambda cc: cc, cnt)

            def app1(cnt):
              qlist[r, jnp.minimum(cnt, _QCAP - 1)] = sid + 1
              return cnt + 1

            return lax.cond(h1, app1, lambda cc: cc, cnt)

          return lax.fori_loop(0, 8, gv_loop, cnt)

        out[r] = lax.cond(hit, do_block, lambda cc: cc, cnt)
      return tuple(out)

    cnts = lax.fori_loop(0, _NBLK, block2,
                         tuple(jnp.int32(0) for _ in range(8)))

    def fslab(sid):
      return proxy.at[pl.ds(rbase, 8), pl.ds(sid * 128, 128)]

    def merge_slab(fb, r, sid, bv, bi):
      for g8 in range(8):
        vals = fb[r, pl.ds(g8 * 16, 16)]
        ghit = jnp.min(vals) < bv[15]

        def do_merge(bv, bi, g8=g8, sid=sid, vals=vals):
          return tuple(_merge16(bv, bi, vals, sid * 128 + g8 * 16 + iota))

        bv, bi = lax.cond(ghit, do_merge, lambda bv, bi: (bv, bi), bv, bi)
      return bv, bi

    for r in range(8):
      cnt = cnts[r]
      of = cnt > _QCAP
      nfetch = jnp.where(of, jnp.int32(_NS), cnt)

      def sid_at(k, r=r, of=of):
        return jnp.where(of, k, qlist[r, jnp.minimum(k, _QCAP - 1)])

      @pl.when(nfetch > 0)
      def _(sid_at=sid_at):
        pltpu.async_copy(fslab(sid_at(0)), fbuf0, sem_f0)

      def fk(k, st, r=r, nfetch=nfetch, sid_at=sid_at):
        bv, bi = st

        @pl.when(k + 1 < nfetch)
        def _():
          nsid = sid_at(k + 1)

          @pl.when((k + 1) % 2 == 0)
          def _():
            pltpu.async_copy(fslab(nsid), fbuf0, sem_f0)

          @pl.when((k + 1) % 2 == 1)
          def _():
            pltpu.async_copy(fslab(nsid), fbuf1, sem_f1)

        sid = sid_at(k)

        def from0(bv, bi):
          pltpu.make_async_copy(fslab(sid), fbuf0, sem_f0).wait()
          return merge_slab(fbuf0, r, sid, bv, bi)

        def from1(bv, bi):
          pltpu.make_async_copy(fslab(sid), fbuf1, sem_f1).wait()
          return merge_slab(fbuf1, r, sid, bv, bi)

        return lax.cond(k % 2 == 0, from0, from1, bv, bi)

      bv, bi = lax.fori_loop(
          0, nfetch, fk,
          (jnp.full((16,), jnp.inf, jnp.float32),
           jnp.zeros((16,), jnp.int32)))
      res_v[r] = bv
      resi_v[r] = bi

    pltpu.sync_copy(res_v, d2_out.at[pl.ds(rbase, 8)])
    pltpu.sync_copy(resi_v, idx_out.at[pl.ds(rbase, 8)])
    return 0

  lax.fori_loop(0, _QPW // 8, octet, 0)


def _topk(proxy, gmin, slack2d):
  mesh = plsc.VectorSubcoreMesh(core_axis_name="c", subcore_axis_name="s")
  f = pl.kernel(
      _topk_body,
      out_type=(
          jax.ShapeDtypeStruct((_M, _K), jnp.float32),
          jax.ShapeDtypeStruct((_M, _K), jnp.int32),
      ),
      mesh=mesh,
      scratch_types=[
          pltpu.VMEM((8, _NG), jnp.float32),
          pltpu.SMEM((8, _QCAP), jnp.int32),
          pltpu.VMEM((8, 128), jnp.float32),
          pltpu.VMEM((8, 128), jnp.float32),
          pltpu.VMEM((8, 128), jnp.float32),
          pltpu.VMEM((8, _K), jnp.float32),
          pltpu.VMEM((8, _K), jnp.int32),
          pltpu.SemaphoreType.DMA,
          pltpu.SemaphoreType.DMA,
          pltpu.SemaphoreType.DMA,
          pltpu.SemaphoreType.DMA,
      ],
      compiler_params=pltpu.CompilerParams(needs_layout_passes=False),
  )
  return f(proxy, gmin, slack2d)


def _sqrt_body(bv_ref, q_ref, out_ref):
  q = q_ref[...]
  qn = jnp.sum(q * q, axis=1, keepdims=True)
  out_ref[...] = jnp.sqrt(jnp.maximum(bv_ref[...] + qn, 0.0))


def _sqrt(bv, q):
  return pl.pallas_call(
      _sqrt_body,
      out_shape=jax.ShapeDtypeStruct((_M, _K), jnp.float32),
  )(bv, q)


def kernel(query, support):
  q = query[0]
  s = support[0]
  qm2 = -2.0 * q
  spad = jnp.pad(s, ((0, _NPAD - _N), (0, 0)), constant_values=_PADVAL)
  sperm = (spad.reshape(_NPAD // _BN, _BN // 16, 16, _D)
           .transpose(0, 2, 1, 3).reshape(_NPAD, _D))
  proxy, gmin = _compute_proxy(qm2, spad, sperm)
  sn_max = jnp.max(jnp.sum(s * s, axis=1))
  qn = jnp.sum(q * q, axis=1)
  slack = 1e-5 * (sn_max + 2.0 * jnp.sqrt(qn * sn_max))
  slack2d = jnp.broadcast_to(slack[:, None], (_M, 128)).astype(jnp.float32)
  bv, idx = _topk(proxy, gmin, slack2d)
  values = _sqrt(bv, q)
  return (values.reshape(1, _M, _K), idx.reshape(1, _M, _K))
